# Initial kernel scaffold; baseline (speedup 1.0000x reference)
import jax
import jax.numpy as jnp
from jax.experimental import pallas as pl


def kernel(node_emb, edge_emb, src, dst):
    raise NotImplementedError("write your pallas kernel here")


# speedup vs baseline: 2.2036x; 2.2036x over previous
import functools

import jax
import jax.numpy as jnp
from jax import lax
from jax.experimental import pallas as pl
from jax.experimental.pallas import tpu as pltpu
from jax.experimental.pallas import tpu_sc as plsc

LANES = 16
NUM_CORES = 2
NUM_SUBCORES = 16
NW = NUM_CORES * NUM_SUBCORES

CHUNK = 80
NBUF = 2


def _dist_mult_body(ew, chunks, dim,
                    node_hbm, edge_hbm, src_hbm, dst_hbm, out_hbm,
                    idx_s, idx_d, head, tail, rel, out_v, sem0, sem1):
    base = (lax.axis_index("s") * NUM_CORES + lax.axis_index("c")) * ew
    sems = (sem0, sem1)

    pltpu.sync_copy(src_hbm.at[pl.ds(base, ew)], idx_s)
    pltpu.sync_copy(dst_hbm.at[pl.ds(base, ew)], idx_d)

    def issue(c, b):
        off = c * CHUNK
        pltpu.async_copy(node_hbm.at[idx_s.at[pl.ds(off, CHUNK)]],
                         head.at[b], sems[b])
        pltpu.async_copy(node_hbm.at[idx_d.at[pl.ds(off, CHUNK)]],
                         tail.at[b], sems[b])
        pltpu.async_copy(edge_hbm.at[pl.ds(base + off, CHUNK)],
                         rel.at[b], sems[b])

    def drain(b):
        for buf in (head, tail, rel):
            pltpu.make_async_copy(edge_hbm.at[pl.ds(0, CHUNK)],
                                  buf.at[b], sems[b]).wait()

    lane = lax.iota(jnp.int32, LANES)

    def compute(c, b):
        off = c * CHUNK

        def group_body(g, _):
            res = jnp.zeros((LANES,), jnp.float32)
            for k in range(LANES):
                e = g * LANES + k
                prods = []
                for j in range(dim // LANES):
                    sl = pl.ds(j * LANES, LANES)
                    prods.append(head[b, e, sl] * rel[b, e, sl]
                                 * tail[b, e, sl])
                while len(prods) > 1:
                    prods = [prods[i] + prods[i + 1]
                             for i in range(0, len(prods), 2)]
                res = jnp.where(lane == k, jnp.sum(prods[0]), res)
            out_v[pl.ds(off + g * LANES, LANES)] = res
            return 0

        lax.fori_loop(0, CHUNK // LANES, group_body, 0)

    for b in range(NBUF):
        issue(b, b)

    def outer(i, _):
        for b in range(NBUF):
            c = i * NBUF + b
            drain(b)
            compute(c, b)

            @pl.when(c + NBUF < chunks)
            def _():
                issue(c + NBUF, b)
        return 0

    lax.fori_loop(0, chunks // NBUF, outer, 0)
    for c in range((chunks // NBUF) * NBUF, chunks):
        drain(c % NBUF)
        compute(c, c % NBUF)

    pltpu.sync_copy(out_v, out_hbm.at[pl.ds(base, ew)])


def kernel(node_emb, edge_emb, src, dst):
    n_edges, dim = edge_emb.shape
    assert n_edges % (NW * CHUNK) == 0 and dim % LANES == 0
    assert CHUNK % LANES == 0
    ew = n_edges // NW
    chunks = ew // CHUNK

    mesh = plsc.VectorSubcoreMesh(core_axis_name="c", subcore_axis_name="s")
    f = pl.kernel(
        functools.partial(_dist_mult_body, ew, chunks, dim),
        out_type=jax.ShapeDtypeStruct((n_edges,), jnp.float32),
        mesh=mesh,
        compiler_params=pltpu.CompilerParams(needs_layout_passes=False),
        scratch_types=[
            pltpu.VMEM((ew,), jnp.int32),
            pltpu.VMEM((ew,), jnp.int32),
            pltpu.VMEM((NBUF, CHUNK, dim), jnp.float32),
            pltpu.VMEM((NBUF, CHUNK, dim), jnp.float32),
            pltpu.VMEM((NBUF, CHUNK, dim), jnp.float32),
            pltpu.VMEM((ew,), jnp.float32),
            pltpu.SemaphoreType.DMA,
            pltpu.SemaphoreType.DMA,
        ],
    )
    return f(node_emb, edge_emb,
             src.astype(jnp.int32), dst.astype(jnp.int32))


# speedup vs baseline: 7.4082x; 3.3618x over previous
import functools

import jax
import jax.numpy as jnp
from jax import lax
from jax.experimental import pallas as pl
from jax.experimental.pallas import tpu as pltpu
from jax.experimental.pallas import tpu_sc as plsc

LANES = 16
NUM_CORES = 2
NUM_SUBCORES = 16
NW = NUM_CORES * NUM_SUBCORES

CHUNK = 80
NBUF = 2


def _dist_mult_body(ew, chunks, dim,
                    node_hbm, edge_hbm, src_hbm, dst_hbm, out_hbm,
                    idx_s, idx_d, head, tail, rel, out_v, sem0, sem1):
    base = (lax.axis_index("s") * NUM_CORES + lax.axis_index("c")) * ew
    sems = (sem0, sem1)

    pltpu.sync_copy(src_hbm.at[pl.ds(base, ew)], idx_s)
    pltpu.sync_copy(dst_hbm.at[pl.ds(base, ew)], idx_d)

    def issue(c, b):
        off = c * CHUNK
        pltpu.async_copy(node_hbm.at[idx_s.at[pl.ds(off, CHUNK)]],
                         head.at[b], sems[b])
        pltpu.async_copy(node_hbm.at[idx_d.at[pl.ds(off, CHUNK)]],
                         tail.at[b], sems[b])
        pltpu.async_copy(edge_hbm.at[pl.ds(base + off, CHUNK)],
                         rel.at[b], sems[b])

    def drain(b):
        for buf in (head, tail, rel):
            pltpu.make_async_copy(edge_hbm.at[pl.ds(0, CHUNK)],
                                  buf.at[b], sems[b]).wait()

    lane = lax.iota(jnp.int32, LANES)
    perms = [lane ^ sh for sh in (1, 2, 4, 8)]
    lane0 = lane == 0

    def compute(c, b):
        off = c * CHUNK

        @plsc.parallel_loop(0, CHUNK, step=1, unroll=4)
        def edge_body(e):
            acc = [None, None]
            for j in range(dim // LANES):
                sl = pl.ds(j * LANES, LANES)
                p = head[b, e, sl] * rel[b, e, sl] * tail[b, e, sl]
                acc[j % 2] = p if acc[j % 2] is None else acc[j % 2] + p
            v = acc[0] + acc[1]
            for p_ix in perms:
                v = v + v.at[p_ix].get(mode="promise_in_bounds")
            plsc.store_compressed(out_v.at[pl.ds(off + e, LANES)], v,
                                  mask=lane0)

    for b in range(NBUF):
        issue(b, b)

    def outer(i, _):
        for b in range(NBUF):
            c = i * NBUF + b
            drain(b)
            compute(c, b)

            @pl.when(c + NBUF < chunks)
            def _():
                issue(c + NBUF, b)
        return 0

    lax.fori_loop(0, chunks // NBUF, outer, 0)
    for c in range((chunks // NBUF) * NBUF, chunks):
        drain(c % NBUF)
        compute(c, c % NBUF)

    pltpu.sync_copy(out_v.at[pl.ds(0, ew)], out_hbm.at[pl.ds(base, ew)])


def kernel(node_emb, edge_emb, src, dst):
    n_edges, dim = edge_emb.shape
    assert n_edges % (NW * CHUNK) == 0 and dim % LANES == 0
    assert CHUNK % LANES == 0
    ew = n_edges // NW
    chunks = ew // CHUNK

    mesh = plsc.VectorSubcoreMesh(core_axis_name="c", subcore_axis_name="s")
    f = pl.kernel(
        functools.partial(_dist_mult_body, ew, chunks, dim),
        out_type=jax.ShapeDtypeStruct((n_edges,), jnp.float32),
        mesh=mesh,
        compiler_params=pltpu.CompilerParams(needs_layout_passes=False),
        scratch_types=[
            pltpu.VMEM((ew,), jnp.int32),
            pltpu.VMEM((ew,), jnp.int32),
            pltpu.VMEM((NBUF, CHUNK, dim), jnp.float32),
            pltpu.VMEM((NBUF, CHUNK, dim), jnp.float32),
            pltpu.VMEM((NBUF, CHUNK, dim), jnp.float32),
            pltpu.VMEM((ew + LANES,), jnp.float32),
            pltpu.SemaphoreType.DMA,
            pltpu.SemaphoreType.DMA,
        ],
    )
    return f(node_emb, edge_emb,
             src.astype(jnp.int32), dst.astype(jnp.int32))


# speedup vs baseline: 9.3099x; 1.2567x over previous
import functools

import jax
import jax.numpy as jnp
from jax import lax
from jax.experimental import pallas as pl
from jax.experimental.pallas import tpu as pltpu
from jax.experimental.pallas import tpu_sc as plsc

LANES = 16
NUM_CORES = 2
NUM_SUBCORES = 16
NW = NUM_CORES * NUM_SUBCORES

CHUNK = 80
NBUF = 2


def _score_body(ew, chunks, dim,
                node_hbm, edge_hbm, src_hbm, dst_hbm, out_hbm,
                idx_s, idx_d, head, tail, rel, out_v, sem0, sem1):
    base = (lax.axis_index("s") * NUM_CORES + lax.axis_index("c")) * ew
    sems = (sem0, sem1)

    pltpu.sync_copy(src_hbm.at[pl.ds(base, ew)], idx_s)
    pltpu.sync_copy(dst_hbm.at[pl.ds(base, ew)], idx_d)

    def issue(c, b):
        off = c * CHUNK
        pltpu.async_copy(node_hbm.at[idx_s.at[pl.ds(off, CHUNK)]],
                         head.at[b], sems[b])
        pltpu.async_copy(node_hbm.at[idx_d.at[pl.ds(off, CHUNK)]],
                         tail.at[b], sems[b])
        pltpu.async_copy(edge_hbm.at[pl.ds(base + off, CHUNK)],
                         rel.at[b], sems[b])

    def drain(b):
        pltpu.make_async_copy(node_hbm.at[pl.ds(0, CHUNK)],
                              head.at[b], sems[b]).wait()
        pltpu.make_async_copy(node_hbm.at[pl.ds(0, CHUNK)],
                              tail.at[b], sems[b]).wait()
        pltpu.make_async_copy(edge_hbm.at[pl.ds(0, CHUNK)],
                              rel.at[b], sems[b]).wait()

    lane = lax.iota(jnp.int32, LANES)
    perms = [lane ^ sh for sh in (1, 2, 4, 8)]
    lane0 = lane == 0

    def compute(c, b):
        off = c * CHUNK

        @plsc.parallel_loop(0, CHUNK, step=1, unroll=4)
        def edge_body(e):
            acc = [None, None]
            for j in range(dim // (2 * LANES)):
                hw = plsc.bitcast(head[b, e, pl.ds(j * LANES, LANES)],
                                  jnp.bfloat16)
                tw = plsc.bitcast(tail[b, e, pl.ds(j * LANES, LANES)],
                                  jnp.bfloat16)
                ht_lo, ht_hi = plsc.unpack(hw * tw,
                                           format=plsc.PackFormat.INTERLEAVED)
                r_lo = rel[b, e, pl.ds(j * 2 * LANES, LANES)]
                r_hi = rel[b, e, pl.ds(j * 2 * LANES + LANES, LANES)]
                p0 = ht_lo * r_lo
                p1 = ht_hi * r_hi
                acc[0] = p0 if acc[0] is None else acc[0] + p0
                acc[1] = p1 if acc[1] is None else acc[1] + p1
            v = acc[0] + acc[1]
            for p_ix in perms:
                v = v + v.at[p_ix].get(mode="promise_in_bounds")
            plsc.store_compressed(out_v.at[pl.ds(off + e, LANES)], v,
                                  mask=lane0)

    for b in range(NBUF):
        issue(b, b)

    def outer(i, _):
        for b in range(NBUF):
            c = i * NBUF + b
            drain(b)
            compute(c, b)

            @pl.when(c + NBUF < chunks)
            def _():
                issue(c + NBUF, b)
        return 0

    lax.fori_loop(0, chunks // NBUF, outer, 0)
    for c in range((chunks // NBUF) * NBUF, chunks):
        drain(c % NBUF)
        compute(c, c % NBUF)

    pltpu.sync_copy(out_v.at[pl.ds(0, ew)], out_hbm.at[pl.ds(base, ew)])


def kernel(node_emb, edge_emb, src, dst):
    n_nodes, dim = node_emb.shape
    n_edges, _ = edge_emb.shape
    assert n_edges % (NW * CHUNK) == 0 and dim % (2 * LANES) == 0
    assert CHUNK % LANES == 0
    ew = n_edges // NW
    chunks = ew // CHUNK

    mesh = plsc.VectorSubcoreMesh(core_axis_name="c", subcore_axis_name="s")
    params = pltpu.CompilerParams(needs_layout_passes=False,
                              use_tc_tiling_on_sc=False)

    node_packed = jax.lax.bitcast_convert_type(
        node_emb.astype(jnp.bfloat16)
        .reshape(n_nodes, dim // 32, 2, 16)
        .swapaxes(2, 3)
        .reshape(n_nodes, dim // 2, 2),
        jnp.int32)

    score = pl.kernel(
        functools.partial(_score_body, ew, chunks, dim),
        out_type=jax.ShapeDtypeStruct((n_edges,), jnp.float32),
        mesh=mesh,
        compiler_params=params,
        scratch_types=[
            pltpu.VMEM((ew,), jnp.int32),
            pltpu.VMEM((ew,), jnp.int32),
            pltpu.VMEM((NBUF, CHUNK, dim // 2), jnp.int32),
            pltpu.VMEM((NBUF, CHUNK, dim // 2), jnp.int32),
            pltpu.VMEM((NBUF, CHUNK, dim), jnp.float32),
            pltpu.VMEM((ew + LANES,), jnp.float32),
            pltpu.SemaphoreType.DMA,
            pltpu.SemaphoreType.DMA,
        ],
    )
    return score(node_packed, edge_emb,
                 src.astype(jnp.int32), dst.astype(jnp.int32))


# speedup vs baseline: 10.5799x; 1.1364x over previous
import functools

import jax
import jax.numpy as jnp
from jax import lax
from jax.experimental import pallas as pl
from jax.experimental.pallas import tpu as pltpu
from jax.experimental.pallas import tpu_sc as plsc

LANES = 16
NUM_CORES = 2
NUM_SUBCORES = 16
NW = NUM_CORES * NUM_SUBCORES

SUPER = 200
GPARTS = (80, 80, 40)
NBUF = 2


def _score_body(ew, supers, dim,
                node_hbm, edge_hbm, src_hbm, dst_hbm, out_hbm,
                idx_s, idx_d, head, tail, rel, outb, sem0, sem1, osem0, osem1):
    wpr = dim // 2
    base = (lax.axis_index("s") * NUM_CORES + lax.axis_index("c")) * ew
    sems = (sem0, sem1)
    osems = (osem0, osem1)

    pltpu.sync_copy(src_hbm.at[pl.ds(base, ew)], idx_s)
    pltpu.sync_copy(dst_hbm.at[pl.ds(base, ew)], idx_d)

    def issue(c, b):
        off = c * SUPER
        part = 0
        for g in GPARTS:
            pltpu.async_copy(node_hbm.at[idx_s.at[pl.ds(off + part, g)]],
                             head.at[b, pl.ds(part, g)], sems[b])
            pltpu.async_copy(node_hbm.at[idx_d.at[pl.ds(off + part, g)]],
                             tail.at[b, pl.ds(part, g)], sems[b])
            part += g
        pltpu.async_copy(edge_hbm.at[pl.ds(base + off, SUPER)],
                         rel.at[b], sems[b])

    def drain(b):
        for g in GPARTS:
            pltpu.make_async_copy(node_hbm.at[pl.ds(0, g)],
                                  head.at[b, pl.ds(0, g)], sems[b]).wait()
            pltpu.make_async_copy(node_hbm.at[pl.ds(0, g)],
                                  tail.at[b, pl.ds(0, g)], sems[b]).wait()
        pltpu.make_async_copy(edge_hbm.at[pl.ds(0, SUPER)],
                              rel.at[b], sems[b]).wait()

    def drain_out(b):
        pltpu.make_async_copy(out_hbm.at[pl.ds(0, SUPER)],
                              outb.at[b, pl.ds(0, SUPER)], osems[b]).wait()

    lane = lax.iota(jnp.int32, LANES)
    perms = [lane ^ sh for sh in (1, 2, 4, 8)]
    lane0 = lane == 0

    def compute(c, b):
        @plsc.parallel_loop(0, SUPER, step=1, unroll=4)
        def edge_body(e):
            acc = [None, None]
            for j in range(wpr // LANES):
                hw = plsc.bitcast(head[b, e, pl.ds(j * LANES, LANES)],
                                  jnp.bfloat16)
                tw = plsc.bitcast(tail[b, e, pl.ds(j * LANES, LANES)],
                                  jnp.bfloat16)
                ht_lo, ht_hi = plsc.unpack(hw * tw,
                                           format=plsc.PackFormat.INTERLEAVED)
                r_lo = rel[b, e, pl.ds(j * 2 * LANES, LANES)]
                r_hi = rel[b, e, pl.ds(j * 2 * LANES + LANES, LANES)]
                p0 = ht_lo * r_lo
                p1 = ht_hi * r_hi
                acc[0] = p0 if acc[0] is None else acc[0] + p0
                acc[1] = p1 if acc[1] is None else acc[1] + p1
            v = acc[0] + acc[1]
            for p_ix in perms:
                v = v + v.at[p_ix].get(mode="promise_in_bounds")
            plsc.store_compressed(outb.at[b, pl.ds(e, LANES)], v, mask=lane0)

    for b in range(NBUF):
        issue(b, b)

    def outer(i, _):
        for b in range(NBUF):
            c = i * NBUF + b
            drain(b)

            @pl.when(c >= NBUF)
            def _():
                drain_out(b)

            compute(c, b)
            pltpu.async_copy(outb.at[b, pl.ds(0, SUPER)],
                             out_hbm.at[pl.ds(base + c * SUPER, SUPER)],
                             osems[b])

            @pl.when(c + NBUF < supers)
            def _():
                issue(c + NBUF, b)
        return 0

    lax.fori_loop(0, supers // NBUF, outer, 0)
    for b in range(NBUF):
        drain_out(b)


def kernel(node_emb, edge_emb, src, dst):
    n_nodes, dim = node_emb.shape
    n_edges, _ = edge_emb.shape
    assert n_edges % (NW * SUPER) == 0 and dim % (2 * LANES) == 0
    assert (n_edges // (NW * SUPER)) % NBUF == 0
    assert sum(GPARTS) == SUPER
    ew = n_edges // NW
    supers = ew // SUPER

    mesh = plsc.VectorSubcoreMesh(core_axis_name="c", subcore_axis_name="s")
    params = pltpu.CompilerParams(needs_layout_passes=False,
                                  use_tc_tiling_on_sc=False)

    node_packed = jax.lax.bitcast_convert_type(
        node_emb.astype(jnp.bfloat16)
        .reshape(n_nodes, dim // 32, 2, 16)
        .swapaxes(2, 3)
        .reshape(n_nodes, dim // 2, 2),
        jnp.int32)

    score = pl.kernel(
        functools.partial(_score_body, ew, supers, dim),
        out_type=jax.ShapeDtypeStruct((n_edges,), jnp.float32),
        mesh=mesh,
        compiler_params=params,
        scratch_types=[
            pltpu.VMEM((ew,), jnp.int32),
            pltpu.VMEM((ew,), jnp.int32),
            pltpu.VMEM((NBUF, SUPER, dim // 2), jnp.int32),
            pltpu.VMEM((NBUF, SUPER, dim // 2), jnp.int32),
            pltpu.VMEM((NBUF, SUPER, dim), jnp.float32),
            pltpu.VMEM((NBUF, SUPER + LANES), jnp.float32),
            pltpu.SemaphoreType.DMA,
            pltpu.SemaphoreType.DMA,
            pltpu.SemaphoreType.DMA,
            pltpu.SemaphoreType.DMA,
        ],
    )
    return score(node_packed, edge_emb,
                 src.astype(jnp.int32), dst.astype(jnp.int32))
